# Initial kernel scaffold; baseline (speedup 1.0000x reference)
import jax
import jax.numpy as jnp
from jax.experimental import pallas as pl


def kernel(entity_emb, user_emb, edge_index, edge_type, weight, norm_item_user_adj, norm_user_neibor, norm_item_neibor, sample_user_item, user_attend_W, user_attend_b, item_kg_gate_W, item_kg_gate_b, item_neibor_gate_W, item_neibor_gate_b, user_item_gate_W, user_item_gate_b, user_neibor_gate_W, user_neibor_gate_b):
    raise NotImplementedError("write your pallas kernel here")


# speedup vs baseline: 5.1525x; 5.1525x over previous
import functools

import jax
import jax.numpy as jnp
from jax import lax
from jax.experimental import pallas as pl
from jax.experimental.pallas import tpu as pltpu
from jax.experimental.pallas import tpu_sc as plsc

N_ITEM = 4096
EMB = 64
QC = 16
N_ENT = 50000
N_USER = 4096
N_EDGE = 800000
N_REL = 16
HIST = 50

NC = 2
NS = 16
NW = NC * NS

EPT = N_EDGE // NS
EC = 1000
NCH = EPT // EC

PAD_ENT = 51200
STRIPE = PAD_ENT // NS

N_UI = N_USER * HIST
UIPT = N_UI // NW
UC = 320
NUC = UIPT // UC

WREP = 128

BM = 256
DIVB = 400


def _sc_agg_body(do_cnt, do_ui, *refs):
  if do_cnt:
    (ent4, tails2, heads, relidx2, wtab, z16, z8, ones8,
     seg_o, cnt_o,
     acc_sh, cnt_sh, idx_t, head_t, ridx_t, rows_t, wrows_t, ones_t,
     sem) = refs
  else:
    (ent4, tails2, heads, relidx2, wtab, z16, samp, ent_full,
     seg_o, ui_o,
     acc_sh, idx_t, head_t, ridx_t, rows_t, wrows_t, sidx_t, urow_t,
     sem) = refs

  c = lax.axis_index("c")
  s = lax.axis_index("s")
  wid = s * NC + c

  row0 = s * STRIPE
  pltpu.sync_copy(z16, acc_sh.at[pl.ds(row0, STRIPE), :])
  if do_cnt:
    pltpu.sync_copy(z8, cnt_sh.at[pl.ds(row0, STRIPE), :])
    pltpu.sync_copy(ones8, ones_t)
  plsc.subcore_barrier()

  ebase = s * EPT

  def chunk(k, carry):
    eb = ebase + k * EC
    pltpu.sync_copy(tails2.at[pl.ds(c * N_EDGE + eb, EC)], idx_t)
    pltpu.sync_copy(heads.at[pl.ds(eb, EC)], head_t)
    pltpu.sync_copy(relidx2.at[pl.ds(c * N_EDGE + eb, EC)], ridx_t)
    pltpu.async_copy(ent4.at[idx_t], rows_t, sem).wait()
    pltpu.async_copy(wtab.at[ridx_t], wrows_t, sem).wait()

    @plsc.parallel_loop(0, EC, 1, unroll=4)
    def mul_body(e):
      rows_t[e, :] = rows_t[e, :] * wrows_t[e, :]

    pltpu.sync_copy(rows_t, acc_sh.at[head_t], add=True)
    if do_cnt:
      pltpu.sync_copy(ones_t, cnt_sh.at[head_t], add=True)
    return carry

  lax.fori_loop(0, NCH, chunk, 0)
  plsc.subcore_barrier()

  pltpu.sync_copy(acc_sh.at[pl.ds(row0, STRIPE), :],
                  seg_o.at[c, pl.ds(row0, STRIPE), :])

  if do_cnt:
    @pl.when(c == 0)
    def _():
      pltpu.sync_copy(cnt_sh.at[pl.ds(row0, STRIPE), :],
                      cnt_o.at[pl.ds(row0, STRIPE), :])

  if do_ui:
    ubase = wid * UIPT

    def uchunk(k, carry):
      b = ubase + k * UC
      pltpu.sync_copy(samp.at[pl.ds(b, UC)], sidx_t)
      pltpu.async_copy(ent_full.at[sidx_t], urow_t, sem).wait()
      pltpu.sync_copy(urow_t, ui_o.at[pl.ds(b, UC), :])
      return carry

    lax.fori_loop(0, NUC, uchunk, 0)


def _sc_launch(do_cnt, do_ui, args):
  mesh = plsc.VectorSubcoreMesh(core_axis_name="c", subcore_axis_name="s",
                                num_cores=NC, num_subcores=NS)
  out_type = [jax.ShapeDtypeStruct((NC, PAD_ENT, QC), jnp.float32)]
  scratch = [pltpu.VMEM_SHARED((PAD_ENT, QC), jnp.float32)]
  if do_cnt:
    out_type.append(jax.ShapeDtypeStruct((PAD_ENT, 8), jnp.float32))
    scratch.append(pltpu.VMEM_SHARED((PAD_ENT, 8), jnp.float32))
  if do_ui:
    out_type.append(jax.ShapeDtypeStruct((N_UI, EMB), jnp.float32))
  scratch += [
      pltpu.VMEM((EC,), jnp.int32),
      pltpu.VMEM((EC,), jnp.int32),
      pltpu.VMEM((EC,), jnp.int32),
      pltpu.VMEM((EC, QC), jnp.float32),
      pltpu.VMEM((EC, QC), jnp.float32),
  ]
  if do_cnt:
    scratch.append(pltpu.VMEM((EC, 8), jnp.float32))
  if do_ui:
    scratch += [pltpu.VMEM((UC,), jnp.int32),
                pltpu.VMEM((UC, EMB), jnp.float32)]
  scratch.append(pltpu.SemaphoreType.DMA)
  f = pl.kernel(
      functools.partial(_sc_agg_body, do_cnt, do_ui),
      out_type=out_type,
      mesh=mesh,
      scratch_types=scratch,
      compiler_params=pltpu.CompilerParams(use_tc_tiling_on_sc=False),
      name="kg_edge_aggregate_sc",
  )
  return f(*args)


def _div_body(sa_ref, sb_ref, cnt_ref, out_ref):
  a = sa_ref[...]
  b = sb_ref[...]
  cnt = cnt_ref[...][:, 0:1]
  kg = jnp.concatenate([a[0], a[1], b[0], b[1]], axis=1)
  out_ref[...] = kg / jnp.maximum(cnt, 1.0)


def _seg_mean(seg_a, seg_b, cnt):
  grid = N_ENT // DIVB
  return pl.pallas_call(
      _div_body,
      grid=(grid,),
      in_specs=[
          pl.BlockSpec((NC, DIVB, QC), lambda i: (0, i, 0)),
          pl.BlockSpec((NC, DIVB, QC), lambda i: (0, i, 0)),
          pl.BlockSpec((DIVB, 8), lambda i: (i, 0)),
      ],
      out_specs=pl.BlockSpec((DIVB, EMB), lambda i: (i, 0)),
      out_shape=jax.ShapeDtypeStruct((N_ENT, EMB), jnp.float32),
  )(seg_a, seg_b, cnt)


def _fused_body(item_adj_ref, user_adj_ref, item_emb_ref, user_emb_ref,
                ue_blk_ref, kg_ref, ui_ref,
                uaW_ref, uab_ref, ikW_ref, ikb_ref, inW_ref, inb_ref,
                ugW_ref, ugb_ref, unW_ref, unb_ref,
                item_out_ref, user_out_ref, reg_ref):
  i = pl.program_id(0)
  dn = (((1,), (1,)), ((), ()))

  agg_nb = jnp.dot(item_adj_ref[...], item_emb_ref[...],
                   preferred_element_type=jnp.float32)
  kg = kg_ref[...]
  gate = jax.nn.sigmoid(
      lax.dot_general(kg, ikW_ref[...], dn,
                      preferred_element_type=jnp.float32) + ikb_ref[...]
      + lax.dot_general(agg_nb, inW_ref[...], dn,
                        preferred_element_type=jnp.float32) + inb_ref[...])
  item_out_ref[...] = gate * kg + (1.0 - gate) * agg_nb

  ucol = jnp.dot(user_adj_ref[...], user_emb_ref[...],
                 preferred_element_type=jnp.float32)
  ui3 = ui_ref[...]
  t = jnp.tanh(
      lax.dot_general(ui3.reshape(HIST * BM, EMB), uaW_ref[...], dn,
                      preferred_element_type=jnp.float32) + uab_ref[...])
  t3 = t.reshape(HIST, BM, EMB)
  ue = ue_blk_ref[...]
  score = jnp.concatenate(
      [jnp.sum(t3[l] * ue, axis=1, keepdims=True) for l in range(HIST)],
      axis=1)
  m = jnp.max(score, axis=1, keepdims=True)
  p = jnp.exp(score - m)
  attn = p / jnp.sum(p, axis=1, keepdims=True)
  agg = jnp.zeros((BM, EMB), jnp.float32)
  for l in range(HIST):
    agg = agg + ui3[l] * attn[:, l:l + 1]
  ugate = jax.nn.sigmoid(
      lax.dot_general(agg, ugW_ref[...], dn,
                      preferred_element_type=jnp.float32) + ugb_ref[...]
      + lax.dot_general(ucol, unW_ref[...], dn,
                        preferred_element_type=jnp.float32) + unb_ref[...])
  user_out_ref[...] = ugate * agg + (1.0 - ugate) * ucol

  @pl.when(i == 0)
  def _():
    reg = (jnp.sum(uaW_ref[...] ** 2) + jnp.sum(uab_ref[...] ** 2)
           + jnp.sum(ikW_ref[...] ** 2) + jnp.sum(ikb_ref[...] ** 2)
           + jnp.sum(inW_ref[...] ** 2) + jnp.sum(inb_ref[...] ** 2)
           + jnp.sum(ugW_ref[...] ** 2) + jnp.sum(ugb_ref[...] ** 2)
           + jnp.sum(unW_ref[...] ** 2) + jnp.sum(unb_ref[...] ** 2))
    reg_ref[...] = reg.reshape(1, 1)


def _fused_dense(item_adj, user_adj, item_emb, user_emb, kg_full, ui,
                 uaW, uab, ikW, ikb, inW, inb, ugW, ugb, unW, unb):
  grid = N_ITEM // BM
  full64 = pl.BlockSpec((EMB, EMB), lambda i: (0, 0))
  bias = pl.BlockSpec((1, EMB), lambda i: (0, 0))
  return pl.pallas_call(
      _fused_body,
      grid=(grid,),
      in_specs=[
          pl.BlockSpec((BM, N_ITEM), lambda i: (i, 0)),
          pl.BlockSpec((BM, N_USER), lambda i: (i, 0)),
          pl.BlockSpec((N_ITEM, EMB), lambda i: (0, 0)),
          pl.BlockSpec((N_USER, EMB), lambda i: (0, 0)),
          pl.BlockSpec((BM, EMB), lambda i: (i, 0)),
          pl.BlockSpec((BM, EMB), lambda i: (i, 0)),
          pl.BlockSpec((HIST, BM, EMB), lambda i: (0, i, 0)),
          full64, bias, full64, bias, full64, bias, full64, bias, full64,
          bias,
      ],
      out_specs=[
          pl.BlockSpec((BM, EMB), lambda i: (i, 0)),
          pl.BlockSpec((BM, EMB), lambda i: (i, 0)),
          pl.BlockSpec((1, 1), lambda i: (0, 0)),
      ],
      out_shape=[
          jax.ShapeDtypeStruct((N_ITEM, EMB), jnp.float32),
          jax.ShapeDtypeStruct((N_USER, EMB), jnp.float32),
          jax.ShapeDtypeStruct((1, 1), jnp.float32),
      ],
  )(item_adj, user_adj, item_emb, user_emb, user_emb, kg_full, ui,
    uaW, uab, ikW, ikb, inW, inb, ugW, ugb, unW, unb)


def kernel(entity_emb, user_emb, edge_index, edge_type, weight,
           norm_item_user_adj, norm_user_neibor, norm_item_neibor,
           sample_user_item,
           user_attend_W, user_attend_b, item_kg_gate_W, item_kg_gate_b,
           item_neibor_gate_W, item_neibor_gate_b, user_item_gate_W,
           user_item_gate_b, user_neibor_gate_W, user_neibor_gate_b):
  del norm_item_user_adj
  head = edge_index[0].astype(jnp.int32)
  tail = edge_index[1].astype(jnp.int32)
  rel = jnp.mod(edge_type.astype(jnp.int32) - 1, N_REL)

  ent4 = jnp.concatenate(
      [entity_emb[:, q * QC:(q + 1) * QC] for q in range(4)], axis=0)
  tails_a = jnp.concatenate([tail, tail + N_ENT])
  tails_b = jnp.concatenate([tail + 2 * N_ENT, tail + 3 * N_ENT])

  w4 = jnp.stack([weight[:, q * QC:(q + 1) * QC] for q in range(4)])
  wtab = jnp.broadcast_to(w4[:, :, None, :],
                          (4, N_REL, WREP, QC)).reshape(-1, QC)
  rep = jnp.arange(N_EDGE, dtype=jnp.int32) % WREP
  rep2 = jnp.concatenate([rep, rep])
  relidx_a = (jnp.concatenate([rel, rel + N_REL]) * WREP + rep2)
  relidx_b = (jnp.concatenate([rel + 2 * N_REL, rel + 3 * N_REL]) * WREP
              + rep2)

  samp = sample_user_item.astype(jnp.int32).T.reshape(-1)
  z16 = jnp.zeros((STRIPE, QC), jnp.float32)
  z8 = jnp.zeros((STRIPE, 8), jnp.float32)
  ones8 = jnp.zeros((EC, 8), jnp.float32).at[:, 0].set(1.0)

  seg_a, cnt = _sc_launch(
      True, False, (ent4, tails_a, head, relidx_a, wtab, z16, z8, ones8))
  seg_b, ui = _sc_launch(
      False, True, (ent4, tails_b, head, relidx_b, wtab, z16, samp,
                    entity_emb))

  kg_full = _seg_mean(seg_a, seg_b, cnt)

  item_emb = lax.slice(entity_emb, (0, 0), (N_ITEM, EMB))
  b = lambda v: v.reshape(1, EMB)
  item_agg, user_agg, reg = _fused_dense(
      norm_item_neibor, norm_user_neibor, item_emb, user_emb, kg_full,
      ui.reshape(HIST, N_USER, EMB),
      user_attend_W, b(user_attend_b), item_kg_gate_W, b(item_kg_gate_b),
      item_neibor_gate_W, b(item_neibor_gate_b), user_item_gate_W,
      b(user_item_gate_b), user_neibor_gate_W, b(user_neibor_gate_b))

  entity_agg = jnp.concatenate(
      [item_agg, lax.slice(kg_full, (N_ITEM, 0), (N_ENT, EMB))], axis=0)
  return entity_agg, user_agg, reg[0, 0]


# speedup vs baseline: 8.4361x; 1.6373x over previous
import functools

import jax
import jax.numpy as jnp
from jax import lax
from jax.experimental import pallas as pl
from jax.experimental.pallas import tpu as pltpu
from jax.experimental.pallas import tpu_sc as plsc

N_ITEM = 4096
EMB = 64
QC = 16
N_ENT = 50000
N_USER = 4096
N_EDGE = 800000
N_REL = 16
HIST = 50

NC = 2
NS = 16
NW = NC * NS

EPT = N_EDGE // NS
EC = 1000
NCH = EPT // EC

PAD_ENT = 51200
STRIPE = PAD_ENT // NS

N_UI = N_USER * HIST
UIPT = N_UI // NW
UC = 640
NUC = UIPT // UC

CPT = N_EDGE // NW
CC = 1000
NCC = CPT // CC

BM = 256
DIVB = 400


_MESH = dict(core_axis_name="c", subcore_axis_name="s",
             num_cores=NC, num_subcores=NS)


def _sc_edge_body(launch, ent4, tails, heads, rels, w4, z16, seg_o,
                  acc_sh, wsh,
                  idx_r0, idx_r1, head_r0, head_r1, rel_r0, rel_r1,
                  idx2_0, idx2_1, ridx2_0, ridx2_1, rows_0, rows_1, wrows,
                  sem_i, sem_g, sem_w):
  idx_r = (idx_r0, idx_r1)
  head_r = (head_r0, head_r1)
  rel_r = (rel_r0, rel_r1)
  idx2 = (idx2_0, idx2_1)
  ridx2 = (ridx2_0, ridx2_1)
  rows = (rows_0, rows_1)

  c = lax.axis_index("c")
  s = lax.axis_index("s")
  q = launch * NC + c

  row0 = s * STRIPE
  pltpu.sync_copy(z16, acc_sh.at[pl.ds(row0, STRIPE), :])

  @pl.when(s == 0)
  def _():
    pltpu.sync_copy(w4.at[pl.ds(q * N_REL, N_REL), :], wsh)

  plsc.subcore_barrier()

  ebase = s * EPT
  nvec = (EC + 15) // 16

  def fire_loads(k, b):
    eb = ebase + k * EC
    pltpu.async_copy(tails.at[pl.ds(eb, EC)], idx_r[b], sem_i)
    pltpu.async_copy(heads.at[pl.ds(eb, EC)], head_r[b], sem_i)
    pltpu.async_copy(rels.at[pl.ds(eb, EC)], rel_r[b], sem_i)

  def wait_loads(b):
    pltpu.make_async_copy(tails.at[pl.ds(0, EC)], idx_r[b], sem_i).wait()
    pltpu.make_async_copy(heads.at[pl.ds(0, EC)], head_r[b], sem_i).wait()
    pltpu.make_async_copy(rels.at[pl.ds(0, EC)], rel_r[b], sem_i).wait()

  def adjust(b):
    def adj(j, carry):
      o = jnp.minimum(j * 16, EC - 16)
      idx2[b][pl.ds(o, 16)] = idx_r[b][pl.ds(o, 16)] * 4 + q
      ridx2[b][pl.ds(o, 16)] = (rel_r[b][pl.ds(o, 16)]
                                + (N_REL - 1)) % N_REL
      return carry
    lax.fori_loop(0, nvec, adj, 0)

  def fire_gathers(b):
    pltpu.async_copy(ent4.at[idx2[b]], rows[b], sem_g)
    pltpu.async_copy(wsh.at[ridx2[b]], wrows, sem_w)

  def wait_gathers(b):
    pltpu.make_async_copy(ent4.at[idx2[b]], rows[b], sem_g).wait()
    pltpu.make_async_copy(wsh.at[ridx2[b]], wrows, sem_w).wait()

  def mult(b):
    @plsc.parallel_loop(0, EC, 1, unroll=8)
    def mul_body(e):
      rows[b][e, :] = rows[b][e, :] * wrows[e, :]

  def scatter(b):
    pltpu.sync_copy(rows[b], acc_sh.at[head_r[b]], add=True)

  fire_loads(0, 0)
  wait_loads(0)
  adjust(0)
  fire_gathers(0)
  fire_loads(1, 1)

  def pair(j, carry):
    for b in (0, 1):
      k = 2 * j + b
      nb = 1 - b
      wait_gathers(b)
      mult(b)

      def prep():
        wait_loads(nb)
        adjust(nb)
        fire_gathers(nb)

      if b == 0:
        prep()
      else:
        pl.when(j < NCH // 2 - 1)(prep)
      scatter(b)
      pl.when(j < NCH // 2 - 1)(lambda: fire_loads(k + 2, b))
    return carry

  lax.fori_loop(0, NCH // 2, pair, 0)
  plsc.subcore_barrier()

  pltpu.sync_copy(acc_sh.at[pl.ds(row0, STRIPE), :],
                  seg_o.at[c, pl.ds(row0, STRIPE), :])


def _sc_cntui_body(heads, samp, item_tab, z8, ones8, cnt_o, ui_o,
                   cnt_sh, ones_t,
                   head_r0, head_r1, sidx_r0, sidx_r1, urow_0, urow_1,
                   sem_i, sem_g, sem_u):
  head_r = (head_r0, head_r1)
  sidx = (sidx_r0, sidx_r1)
  urow = (urow_0, urow_1)

  c = lax.axis_index("c")
  s = lax.axis_index("s")
  wid = s * NC + c

  row0 = s * STRIPE
  pltpu.sync_copy(z8, cnt_sh.at[pl.ds(row0, STRIPE), :])
  pltpu.sync_copy(ones8, ones_t)
  plsc.subcore_barrier()

  cbase = wid * CPT

  def fire_head(k, b):
    pltpu.async_copy(heads.at[pl.ds(cbase + k * CC, CC)], head_r[b], sem_i)

  def wait_head(b):
    pltpu.make_async_copy(heads.at[pl.ds(0, CC)], head_r[b], sem_i).wait()

  fire_head(0, 0)
  fire_head(1, 1)

  def cpair(j, carry):
    for b in (0, 1):
      k = 2 * j + b

      def step():
        wait_head(b)
        pltpu.sync_copy(ones_t, cnt_sh.at[head_r[b]], add=True)

      if b == 0:
        step()
        pl.when(j < 12)(lambda: fire_head(k + 2, b))
      else:
        pl.when(j < 12)(step)
        pl.when(j < 11)(lambda: fire_head(k + 2, b))
    return carry

  lax.fori_loop(0, (NCC + 1) // 2, cpair, 0)
  plsc.subcore_barrier()
  pltpu.sync_copy(cnt_sh.at[pl.ds(row0, STRIPE), :],
                  cnt_o.at[c, pl.ds(row0, STRIPE), :])

  ubase = wid * UIPT

  def fire_sidx(k, b):
    pltpu.async_copy(samp.at[pl.ds(ubase + k * UC, UC)], sidx[b], sem_u)

  def wait_sidx(b):
    pltpu.make_async_copy(samp.at[pl.ds(0, UC)], sidx[b], sem_u).wait()

  def fire_ug(b):
    pltpu.async_copy(item_tab.at[sidx[b]], urow[b], sem_g)

  def wait_ug(b):
    pltpu.make_async_copy(item_tab.at[sidx[b]], urow[b], sem_g).wait()

  fire_sidx(0, 0)
  wait_sidx(0)
  fire_ug(0)
  fire_sidx(1, 1)

  def upair(j, carry):
    for b in (0, 1):
      k = 2 * j + b
      nb = 1 - b
      wait_ug(b)

      def prep():
        wait_sidx(nb)
        fire_ug(nb)

      if b == 0:
        prep()
      else:
        pl.when(j < NUC // 2 - 1)(prep)
      pltpu.sync_copy(urow[b], ui_o.at[pl.ds(ubase + k * UC, UC), :])
      pl.when(j < NUC // 2 - 1)(lambda: fire_sidx(k + 2, b))
    return carry

  lax.fori_loop(0, NUC // 2, upair, 0)


def _sc_edge(launch, args):
  mesh = plsc.VectorSubcoreMesh(**_MESH)
  f = pl.kernel(
      functools.partial(_sc_edge_body, launch),
      out_type=[jax.ShapeDtypeStruct((NC, PAD_ENT, QC), jnp.float32)],
      mesh=mesh,
      scratch_types=[
          pltpu.VMEM_SHARED((PAD_ENT, QC), jnp.float32),
          pltpu.VMEM_SHARED((N_REL, QC), jnp.float32),
      ] + [pltpu.VMEM((EC,), jnp.int32)] * 10 + [
          pltpu.VMEM((EC, QC), jnp.float32),
          pltpu.VMEM((EC, QC), jnp.float32),
          pltpu.VMEM((EC, QC), jnp.float32),
          pltpu.SemaphoreType.DMA,
          pltpu.SemaphoreType.DMA,
          pltpu.SemaphoreType.DMA,
      ],
      compiler_params=pltpu.CompilerParams(use_tc_tiling_on_sc=False),
      name="kg_edge_aggregate_sc",
  )
  return f(*args)


def _sc_cntui(args):
  mesh = plsc.VectorSubcoreMesh(**_MESH)
  f = pl.kernel(
      _sc_cntui_body,
      out_type=[
          jax.ShapeDtypeStruct((NC, PAD_ENT, 8), jnp.float32),
          jax.ShapeDtypeStruct((N_UI, EMB), jnp.float32),
      ],
      mesh=mesh,
      scratch_types=[
          pltpu.VMEM_SHARED((PAD_ENT, 8), jnp.float32),
          pltpu.VMEM((CC, 8), jnp.float32),
          pltpu.VMEM((CC,), jnp.int32),
          pltpu.VMEM((CC,), jnp.int32),
          pltpu.VMEM((UC,), jnp.int32),
          pltpu.VMEM((UC,), jnp.int32),
          pltpu.VMEM((UC, EMB), jnp.float32),
          pltpu.VMEM((UC, EMB), jnp.float32),
          pltpu.SemaphoreType.DMA,
          pltpu.SemaphoreType.DMA,
          pltpu.SemaphoreType.DMA,
      ],
      compiler_params=pltpu.CompilerParams(use_tc_tiling_on_sc=False),
      name="kg_count_ui_sc",
  )
  return f(*args)


def _div_body(sa_ref, sb_ref, cnt_ref, out_ref):
  a = sa_ref[...]
  b = sb_ref[...]
  cnt = cnt_ref[...]
  csum = cnt[0][:, 0:1] + cnt[1][:, 0:1]
  kg = jnp.concatenate([a[0], a[1], b[0], b[1]], axis=1)
  out_ref[...] = kg / jnp.maximum(csum, 1.0)


def _seg_mean(seg_a, seg_b, cnt):
  grid = N_ENT // DIVB
  return pl.pallas_call(
      _div_body,
      grid=(grid,),
      in_specs=[
          pl.BlockSpec((NC, DIVB, QC), lambda i: (0, i, 0)),
          pl.BlockSpec((NC, DIVB, QC), lambda i: (0, i, 0)),
          pl.BlockSpec((NC, DIVB, 8), lambda i: (0, i, 0)),
      ],
      out_specs=pl.BlockSpec((DIVB, EMB), lambda i: (i, 0)),
      out_shape=jax.ShapeDtypeStruct((N_ENT, EMB), jnp.float32),
  )(seg_a, seg_b, cnt)


def _mm_body(item_adj_ref, user_adj_ref, item_emb_ref, user_emb_ref,
             aggn_out_ref, ucol_out_ref):
  aggn_out_ref[...] = jnp.dot(item_adj_ref[...], item_emb_ref[...],
                              preferred_element_type=jnp.float32)
  ucol_out_ref[...] = jnp.dot(user_adj_ref[...], user_emb_ref[...],
                              preferred_element_type=jnp.float32)


def _adj_matmuls(item_adj, user_adj, item_emb, user_emb):
  grid = N_ITEM // BM
  return pl.pallas_call(
      _mm_body,
      grid=(grid,),
      in_specs=[
          pl.BlockSpec((BM, N_ITEM), lambda i: (i, 0)),
          pl.BlockSpec((BM, N_USER), lambda i: (i, 0)),
          pl.BlockSpec((N_ITEM, EMB), lambda i: (0, 0)),
          pl.BlockSpec((N_USER, EMB), lambda i: (0, 0)),
      ],
      out_specs=[
          pl.BlockSpec((BM, EMB), lambda i: (i, 0)),
          pl.BlockSpec((BM, EMB), lambda i: (i, 0)),
      ],
      out_shape=[
          jax.ShapeDtypeStruct((N_ITEM, EMB), jnp.float32),
          jax.ShapeDtypeStruct((N_USER, EMB), jnp.float32),
      ],
  )(item_adj, user_adj, item_emb, user_emb)


def _fused_body(agg_nb_ref, ucol_ref, ue_blk_ref, kg_ref, ui_ref,
                uaW_ref, uab_ref, ikW_ref, ikb_ref, inW_ref, inb_ref,
                ugW_ref, ugb_ref, unW_ref, unb_ref,
                item_out_ref, user_out_ref, reg_ref):
  i = pl.program_id(0)
  dn = (((1,), (1,)), ((), ()))

  agg_nb = agg_nb_ref[...]
  kg = kg_ref[...]
  gate = jax.nn.sigmoid(
      lax.dot_general(kg, ikW_ref[...], dn,
                      preferred_element_type=jnp.float32) + ikb_ref[...]
      + lax.dot_general(agg_nb, inW_ref[...], dn,
                        preferred_element_type=jnp.float32) + inb_ref[...])
  item_out_ref[...] = gate * kg + (1.0 - gate) * agg_nb

  ucol = ucol_ref[...]
  ui3 = ui_ref[...]
  t = jnp.tanh(
      lax.dot_general(ui3.reshape(HIST * BM, EMB), uaW_ref[...], dn,
                      preferred_element_type=jnp.float32) + uab_ref[...])
  t3 = t.reshape(HIST, BM, EMB)
  ue = ue_blk_ref[...]
  score = jnp.concatenate(
      [jnp.sum(t3[l] * ue, axis=1, keepdims=True) for l in range(HIST)],
      axis=1)
  m = jnp.max(score, axis=1, keepdims=True)
  p = jnp.exp(score - m)
  attn = p / jnp.sum(p, axis=1, keepdims=True)
  agg = jnp.zeros((BM, EMB), jnp.float32)
  for l in range(HIST):
    agg = agg + ui3[l] * attn[:, l:l + 1]
  ugate = jax.nn.sigmoid(
      lax.dot_general(agg, ugW_ref[...], dn,
                      preferred_element_type=jnp.float32) + ugb_ref[...]
      + lax.dot_general(ucol, unW_ref[...], dn,
                        preferred_element_type=jnp.float32) + unb_ref[...])
  user_out_ref[...] = ugate * agg + (1.0 - ugate) * ucol

  @pl.when(i == 0)
  def _():
    reg = (jnp.sum(uaW_ref[...] ** 2) + jnp.sum(uab_ref[...] ** 2)
           + jnp.sum(ikW_ref[...] ** 2) + jnp.sum(ikb_ref[...] ** 2)
           + jnp.sum(inW_ref[...] ** 2) + jnp.sum(inb_ref[...] ** 2)
           + jnp.sum(ugW_ref[...] ** 2) + jnp.sum(ugb_ref[...] ** 2)
           + jnp.sum(unW_ref[...] ** 2) + jnp.sum(unb_ref[...] ** 2))
    reg_ref[...] = reg.reshape(1, 1)


def _fused_dense(agg_nb, ucol, user_emb, kg_full, ui,
                 uaW, uab, ikW, ikb, inW, inb, ugW, ugb, unW, unb):
  grid = N_ITEM // BM
  blk = pl.BlockSpec((BM, EMB), lambda i: (i, 0))
  full64 = pl.BlockSpec((EMB, EMB), lambda i: (0, 0))
  bias = pl.BlockSpec((1, EMB), lambda i: (0, 0))
  return pl.pallas_call(
      _fused_body,
      grid=(grid,),
      in_specs=[
          blk, blk, blk, blk,
          pl.BlockSpec((HIST, BM, EMB), lambda i: (0, i, 0)),
          full64, bias, full64, bias, full64, bias, full64, bias, full64,
          bias,
      ],
      out_specs=[
          pl.BlockSpec((BM, EMB), lambda i: (i, 0)),
          pl.BlockSpec((BM, EMB), lambda i: (i, 0)),
          pl.BlockSpec((1, 1), lambda i: (0, 0)),
      ],
      out_shape=[
          jax.ShapeDtypeStruct((N_ITEM, EMB), jnp.float32),
          jax.ShapeDtypeStruct((N_USER, EMB), jnp.float32),
          jax.ShapeDtypeStruct((1, 1), jnp.float32),
      ],
  )(agg_nb, ucol, user_emb, kg_full, ui,
    uaW, uab, ikW, ikb, inW, inb, ugW, ugb, unW, unb)


def kernel(entity_emb, user_emb, edge_index, edge_type, weight,
           norm_item_user_adj, norm_user_neibor, norm_item_neibor,
           sample_user_item,
           user_attend_W, user_attend_b, item_kg_gate_W, item_kg_gate_b,
           item_neibor_gate_W, item_neibor_gate_b, user_item_gate_W,
           user_item_gate_b, user_neibor_gate_W, user_neibor_gate_b):
  del norm_item_user_adj
  head = edge_index[0].astype(jnp.int32)
  tail = edge_index[1].astype(jnp.int32)
  rels = edge_type.astype(jnp.int32)

  ent4 = entity_emb.reshape(4 * N_ENT, QC)
  w4 = weight.reshape(N_REL, 4, QC).transpose(1, 0, 2).reshape(-1, QC)

  samp = sample_user_item.astype(jnp.int32).T.reshape(-1)
  z16 = jnp.zeros((STRIPE, QC), jnp.float32)
  z8 = jnp.zeros((STRIPE, 8), jnp.float32)
  ones8 = jnp.zeros((CC, 8), jnp.float32).at[:, 0].set(1.0)

  item_emb = lax.slice(entity_emb, (0, 0), (N_ITEM, EMB))

  cnt, ui = _sc_cntui((head, samp, item_emb, z8, ones8))
  seg_a, = _sc_edge(0, (ent4, tail, head, rels, w4, z16))
  seg_b, = _sc_edge(1, (ent4, tail, head, rels, w4, z16))

  agg_nb, ucol = _adj_matmuls(norm_item_neibor, norm_user_neibor,
                              item_emb, user_emb)

  kg_full = _seg_mean(seg_a, seg_b, cnt)

  b = lambda v: v.reshape(1, EMB)
  item_agg, user_agg, reg = _fused_dense(
      agg_nb, ucol, user_emb, kg_full,
      ui.reshape(HIST, N_USER, EMB),
      user_attend_W, b(user_attend_b), item_kg_gate_W, b(item_kg_gate_b),
      item_neibor_gate_W, b(item_neibor_gate_b), user_item_gate_W,
      b(user_item_gate_b), user_neibor_gate_W, b(user_neibor_gate_b))

  entity_agg = jnp.concatenate(
      [item_agg, lax.slice(kg_full, (N_ITEM, 0), (N_ENT, EMB))], axis=0)
  return entity_agg, user_agg, reg[0, 0]
